# baseline (speedup 1.0000x reference)
import jax
import jax.numpy as jnp
from jax import lax
from jax.experimental import pallas as pl
from jax.experimental.pallas import tpu as pltpu
from jax.experimental.pallas import tpu_sc as plsc

_MAX_DIST = 10.0
_GRID_RESOLUTION = 1.0
_BOX = 21
_ZPAD = 24
_NC = 2
_NS = 16
_ROWS_PER_SUB = 672
_ACC_ROWS = _ROWS_PER_SUB * _NS
_TRASH = _ACC_ROWS - 1
_CHUNK = 128


def _voxelize_body(ct_ref, vox_ref):
    x = ct_ref[0]
    y = ct_ref[1]
    z = ct_ref[2]
    gx = jnp.round((x + _MAX_DIST) / _GRID_RESOLUTION)
    gy = jnp.round((y + _MAX_DIST) / _GRID_RESOLUTION)
    gz = jnp.round((z + _MAX_DIST) / _GRID_RESOLUTION)
    hi = float(_BOX - 1)
    inb = ((gx >= 0.0) & (gx <= hi) & (gy >= 0.0) & (gy <= hi)
           & (gz >= 0.0) & (gz <= hi))
    gxc = jnp.clip(gx, 0.0, hi)
    gyc = jnp.clip(gy, 0.0, hi)
    gzc = jnp.clip(gz, 0.0, hi)
    v = (gxc * float(_BOX * _ZPAD) + gyc * float(_ZPAD) + gzc).astype(jnp.int32)
    vox_ref[...] = jnp.where(inb, v, _TRASH)


def _voxelize(coords3):
    _, r, c = coords3.shape
    return pl.pallas_call(
        _voxelize_body,
        out_shape=jax.ShapeDtypeStruct((r, c), jnp.int32),
    )(coords3)


def _scatter_body(vox_hbm, feats_hbm, zeros_hbm, out_hbm,
                  idx_v, feat2_v, feat_v, acc_sh, sem_i, sem0, sem1):
    c = lax.axis_index("c")
    s = lax.axis_index("s")
    tile = c * _NS + s
    n_chunks = idx_v.shape[0]
    f = feat_v.shape[2]
    half = _CHUNK // 2
    base0 = tile * n_chunks * half
    sems = (sem0, sem1)

    h_idx = pltpu.async_copy(vox_hbm.at[pl.ds(tile * n_chunks, n_chunks)],
                             idx_v, sem_i)
    loads = [None] * n_chunks
    for j in range(min(2, n_chunks)):
        loads[j] = pltpu.async_copy(
            feats_hbm.at[pl.ds(base0 + j * half, half)],
            feat2_v.at[j % 2], sems[j % 2])

    pltpu.sync_copy(zeros_hbm,
                    acc_sh.at[pl.ds(s * _ROWS_PER_SUB, _ROWS_PER_SUB)])
    plsc.subcore_barrier()
    h_idx.wait()

    groups = f // 16

    for j in range(n_chunks):
        loads[j].wait()
        src = feat2_v.at[j % 2]
        dst = feat_v.at[j % 2]

        def _row(i, carry, src=src, dst=dst):
            for j0 in range(2 * groups):
                v = src[i, pl.ds(16 * j0, 16)]
                dst[2 * i + j0 // groups, pl.ds(16 * (j0 % groups), 16)] = v
            return carry

        lax.fori_loop(0, half, _row, 0)
        pltpu.sync_copy(dst, acc_sh.at[idx_v.at[j]], add=True)
        nxt = j + 2
        if nxt < n_chunks:
            loads[nxt] = pltpu.async_copy(
                feats_hbm.at[pl.ds(base0 + nxt * half, half)],
                feat2_v.at[nxt % 2], sems[nxt % 2])

    plsc.subcore_barrier()
    pltpu.sync_copy(acc_sh.at[pl.ds(s * _ROWS_PER_SUB, _ROWS_PER_SUB)],
                    out_hbm.at[c, pl.ds(s * _ROWS_PER_SUB, _ROWS_PER_SUB)])


def _scatter(vox2d, feats2, zeros, f):
    n_chunks = vox2d.shape[0] // (_NC * _NS)
    mesh = plsc.VectorSubcoreMesh(core_axis_name="c", subcore_axis_name="s")
    k = pl.kernel(
        _scatter_body,
        out_type=jax.ShapeDtypeStruct((_NC, _ACC_ROWS, f), jnp.float32),
        mesh=mesh,
        compiler_params=pltpu.CompilerParams(use_tc_tiling_on_sc=False),
        scratch_types=[
            pltpu.VMEM((n_chunks, _CHUNK), jnp.int32),
            pltpu.VMEM((2, _CHUNK // 2, 2 * f), jnp.float32),
            pltpu.VMEM((2, _CHUNK, f), jnp.float32),
            pltpu.VMEM_SHARED((_ACC_ROWS, f), jnp.float32),
            pltpu.SemaphoreType.DMA,
            pltpu.SemaphoreType.DMA,
            pltpu.SemaphoreType.DMA,
        ],
    )
    return k(vox2d, feats2, zeros)


def _combine_body(p_ref, out_ref):
    v = p_ref[0] + p_ref[1]
    out_ref[...] = v[None, :, :, :_BOX, :]


def _combine(partials5):
    f = partials5.shape[4]
    return pl.pallas_call(
        _combine_body,
        out_shape=jax.ShapeDtypeStruct((1, _BOX, _BOX, _BOX, f), jnp.float32),
    )(partials5)


def kernel(coords, features):
    b, n, _ = coords.shape
    f = features.shape[2]
    p = b * n
    assert p % (_NC * _NS * _CHUNK) == 0

    coords3 = coords.reshape(p // _CHUNK, _CHUNK, 3).transpose(2, 0, 1)
    vox2d = _voxelize(coords3)
    feats2 = features.reshape(p // 2, 2 * f)
    zeros = jnp.zeros((_ROWS_PER_SUB, f), jnp.float32)
    partials = _scatter(vox2d, feats2, zeros, f)
    partials5 = partials[:, :_BOX * _BOX * _ZPAD].reshape(
        _NC, _BOX, _BOX, _ZPAD, f)
    batch0 = _combine(partials5)
    grid = jnp.zeros((b, _BOX, _BOX, _BOX, f), jnp.float32)
    return lax.dynamic_update_slice(grid, batch0, (0, 0, 0, 0, 0))


# speedup vs baseline: 1.1355x; 1.1355x over previous
import jax
import jax.numpy as jnp
from jax import lax
from jax.experimental import pallas as pl
from jax.experimental.pallas import tpu as pltpu
from jax.experimental.pallas import tpu_sc as plsc

_MAX_DIST = 10.0
_GRID_RESOLUTION = 1.0
_BOX = 21
_ZPAD = 24
_NC = 2
_NS = 16
_ROWS_PER_SUB = 672
_ACC_ROWS = _ROWS_PER_SUB * _NS
_TRASH = _ACC_ROWS - 1
_CHUNK = 128


def _voxelize_body(ct_ref, vox_ref):
    x = ct_ref[0]
    y = ct_ref[1]
    z = ct_ref[2]
    gx = jnp.round((x + _MAX_DIST) / _GRID_RESOLUTION)
    gy = jnp.round((y + _MAX_DIST) / _GRID_RESOLUTION)
    gz = jnp.round((z + _MAX_DIST) / _GRID_RESOLUTION)
    hi = float(_BOX - 1)
    inb = ((gx >= 0.0) & (gx <= hi) & (gy >= 0.0) & (gy <= hi)
           & (gz >= 0.0) & (gz <= hi))
    gxc = jnp.clip(gx, 0.0, hi)
    gyc = jnp.clip(gy, 0.0, hi)
    gzc = jnp.clip(gz, 0.0, hi)
    v = (gxc * float(_BOX * _ZPAD) + gyc * float(_ZPAD) + gzc).astype(jnp.int32)
    vox_ref[...] = jnp.where(inb, v, _TRASH)


def _voxelize(coords3):
    _, r, c = coords3.shape
    return pl.pallas_call(
        _voxelize_body,
        out_shape=jax.ShapeDtypeStruct((r, c), jnp.int32),
    )(coords3)


_OUT_ROWS = _BOX * _BOX * _ZPAD


def _scatter_body(vox_hbm, feats_hbm, out_hbm,
                  idx_v, feat2_v, feat_v, acc_sh, sem_i, sem0, sem1):
    c = lax.axis_index("c")
    s = lax.axis_index("s")
    tile = c * _NS + s
    n_chunks = idx_v.shape[0]
    f = feat_v.shape[2]
    half = _CHUNK // 2
    base0 = tile * n_chunks * half
    sems = (sem0, sem1)

    h_idx = pltpu.async_copy(vox_hbm.at[pl.ds(tile * n_chunks, n_chunks)],
                             idx_v, sem_i)
    loads = [None] * n_chunks
    for j in range(min(2, n_chunks)):
        loads[j] = pltpu.async_copy(
            feats_hbm.at[pl.ds(base0 + j * half, half)],
            feat2_v.at[j % 2], sems[j % 2])

    zsrc = feat_v.at[0]

    def _zrow(i, carry):
        for q in range(f // 16):
            zsrc[i, pl.ds(16 * q, 16)] = jnp.zeros((16,), jnp.float32)
        return carry

    lax.fori_loop(0, _CHUNK, _zrow, 0)
    sbase = s * _ROWS_PER_SUB
    for kk in range(_ROWS_PER_SUB // _CHUNK):
        pltpu.sync_copy(zsrc, acc_sh.at[pl.ds(sbase + kk * _CHUNK, _CHUNK)])
    rem = _ROWS_PER_SUB % _CHUNK
    if rem:
        pltpu.sync_copy(
            zsrc.at[pl.ds(0, rem)],
            acc_sh.at[pl.ds(sbase + _ROWS_PER_SUB - rem, rem)])
    plsc.subcore_barrier()
    h_idx.wait()

    groups = f // 16

    for j in range(n_chunks):
        loads[j].wait()
        src = feat2_v.at[j % 2]
        dst = feat_v.at[j % 2]

        def _row(i, carry, src=src, dst=dst):
            for j0 in range(2 * groups):
                v = src[i, pl.ds(16 * j0, 16)]
                dst[2 * i + j0 // groups, pl.ds(16 * (j0 % groups), 16)] = v
            return carry

        lax.fori_loop(0, half, _row, 0)
        pltpu.sync_copy(dst, acc_sh.at[idx_v.at[j]], add=True)
        nxt = j + 2
        if nxt < n_chunks:
            loads[nxt] = pltpu.async_copy(
                feats_hbm.at[pl.ds(base0 + nxt * half, half)],
                feat2_v.at[nxt % 2], sems[nxt % 2])

    plsc.subcore_barrier()

    last = _OUT_ROWS - (_NS - 1) * _ROWS_PER_SUB

    @pl.when(s < _NS - 1)
    def _():
        pltpu.sync_copy(
            acc_sh.at[pl.ds(s * _ROWS_PER_SUB, _ROWS_PER_SUB)],
            out_hbm.at[c, pl.ds(s * _ROWS_PER_SUB, _ROWS_PER_SUB)])

    @pl.when(s == _NS - 1)
    def _():
        pltpu.sync_copy(
            acc_sh.at[pl.ds((_NS - 1) * _ROWS_PER_SUB, last)],
            out_hbm.at[c, pl.ds((_NS - 1) * _ROWS_PER_SUB, last)])


def _scatter(vox2d, feats2, f):
    n_chunks = vox2d.shape[0] // (_NC * _NS)
    mesh = plsc.VectorSubcoreMesh(core_axis_name="c", subcore_axis_name="s")
    k = pl.kernel(
        _scatter_body,
        out_type=jax.ShapeDtypeStruct((_NC, _OUT_ROWS, f), jnp.float32),
        mesh=mesh,
        compiler_params=pltpu.CompilerParams(use_tc_tiling_on_sc=False),
        scratch_types=[
            pltpu.VMEM((n_chunks, _CHUNK), jnp.int32),
            pltpu.VMEM((2, _CHUNK // 2, 2 * f), jnp.float32),
            pltpu.VMEM((2, _CHUNK, f), jnp.float32),
            pltpu.VMEM_SHARED((_ACC_ROWS, f), jnp.float32),
            pltpu.SemaphoreType.DMA,
            pltpu.SemaphoreType.DMA,
            pltpu.SemaphoreType.DMA,
        ],
    )
    return k(vox2d, feats2)


def _combine_body(p_ref, out_ref):
    v = p_ref[0] + p_ref[1]
    out_ref[...] = v[None, :, :, :_BOX, :]


def _combine(partials5):
    f = partials5.shape[4]
    return pl.pallas_call(
        _combine_body,
        out_shape=jax.ShapeDtypeStruct((1, _BOX, _BOX, _BOX, f), jnp.float32),
    )(partials5)


def kernel(coords, features):
    b, n, _ = coords.shape
    f = features.shape[2]
    p = b * n
    assert p % (_NC * _NS * _CHUNK) == 0

    coords3 = coords.reshape(p // _CHUNK, _CHUNK, 3).transpose(2, 0, 1)
    vox2d = _voxelize(coords3)
    feats2 = features.reshape(p // 2, 2 * f)
    partials = _scatter(vox2d, feats2, f)
    partials5 = partials.reshape(_NC, _BOX, _BOX, _ZPAD, f)
    batch0 = _combine(partials5)
    grid = jnp.zeros((b, _BOX, _BOX, _BOX, f), jnp.float32)
    return lax.dynamic_update_slice(grid, batch0, (0, 0, 0, 0, 0))


# speedup vs baseline: 1.1488x; 1.0117x over previous
import jax
import jax.numpy as jnp
from jax import lax
from jax.experimental import pallas as pl
from jax.experimental.pallas import tpu as pltpu
from jax.experimental.pallas import tpu_sc as plsc

_MAX_DIST = 10.0
_GRID_RESOLUTION = 1.0
_BOX = 21
_ZPAD = 24
_NC = 2
_NS = 16
_ROWS_PER_SUB = 672
_ACC_ROWS = _ROWS_PER_SUB * _NS
_TRASH = _ACC_ROWS - 1
_CHUNK = 128


def _voxelize_body(ct_ref, vox_ref):
    x = ct_ref[0]
    y = ct_ref[1]
    z = ct_ref[2]
    gx = jnp.round((x + _MAX_DIST) / _GRID_RESOLUTION)
    gy = jnp.round((y + _MAX_DIST) / _GRID_RESOLUTION)
    gz = jnp.round((z + _MAX_DIST) / _GRID_RESOLUTION)
    hi = float(_BOX - 1)
    inb = ((gx >= 0.0) & (gx <= hi) & (gy >= 0.0) & (gy <= hi)
           & (gz >= 0.0) & (gz <= hi))
    gxc = jnp.clip(gx, 0.0, hi)
    gyc = jnp.clip(gy, 0.0, hi)
    gzc = jnp.clip(gz, 0.0, hi)
    v = (gxc * float(_BOX * _ZPAD) + gyc * float(_ZPAD) + gzc).astype(jnp.int32)
    vox_ref[...] = jnp.where(inb, v, _TRASH)


def _voxelize(coords3):
    _, r, c = coords3.shape
    return pl.pallas_call(
        _voxelize_body,
        out_shape=jax.ShapeDtypeStruct((r, c), jnp.int32),
    )(coords3)


_OUT_ROWS = _BOX * _BOX * _ZPAD


def _scatter_body(vox_hbm, feats_hbm, out_hbm,
                  idx_v, zbuf, feat_v, acc_sh, sem_i, sem0, sem1):
    c = lax.axis_index("c")
    s = lax.axis_index("s")
    tile = c * _NS + s
    n_chunks = idx_v.shape[0]
    f = feat_v.shape[2]
    rows_b = feats_hbm.shape[1]
    base0 = tile * n_chunks * _CHUNK
    sems = (sem0, sem1)

    def _feat_load(j, buf):
        p0 = base0 + j * _CHUNK
        return pltpu.async_copy(
            feats_hbm.at[p0 // rows_b, pl.ds(p0 % rows_b, _CHUNK)],
            feat_v.at[buf], sems[buf])

    h_idx = pltpu.async_copy(vox_hbm.at[pl.ds(tile * n_chunks, n_chunks)],
                             idx_v, sem_i)
    loads = [None] * n_chunks
    for j in range(min(2, n_chunks)):
        loads[j] = _feat_load(j, j % 2)

    def _zrow(i, carry):
        for q in range(f // 16):
            zbuf[i, pl.ds(16 * q, 16)] = jnp.zeros((16,), jnp.float32)
        return carry

    lax.fori_loop(0, _CHUNK, _zrow, 0)
    sbase = s * _ROWS_PER_SUB
    for kk in range(_ROWS_PER_SUB // _CHUNK):
        pltpu.sync_copy(zbuf, acc_sh.at[pl.ds(sbase + kk * _CHUNK, _CHUNK)])
    rem = _ROWS_PER_SUB % _CHUNK
    if rem:
        pltpu.sync_copy(
            zbuf.at[pl.ds(0, rem)],
            acc_sh.at[pl.ds(sbase + _ROWS_PER_SUB - rem, rem)])
    plsc.subcore_barrier()
    h_idx.wait()

    for j in range(n_chunks):
        loads[j].wait()
        pltpu.sync_copy(feat_v.at[j % 2], acc_sh.at[idx_v.at[j]], add=True)
        nxt = j + 2
        if nxt < n_chunks:
            loads[nxt] = _feat_load(nxt, nxt % 2)

    plsc.subcore_barrier()

    last = _OUT_ROWS - (_NS - 1) * _ROWS_PER_SUB

    @pl.when(s < _NS - 1)
    def _():
        pltpu.sync_copy(
            acc_sh.at[pl.ds(s * _ROWS_PER_SUB, _ROWS_PER_SUB)],
            out_hbm.at[c, pl.ds(s * _ROWS_PER_SUB, _ROWS_PER_SUB)])

    @pl.when(s == _NS - 1)
    def _():
        pltpu.sync_copy(
            acc_sh.at[pl.ds((_NS - 1) * _ROWS_PER_SUB, last)],
            out_hbm.at[c, pl.ds((_NS - 1) * _ROWS_PER_SUB, last)])


def _scatter(vox2d, feats2, f):
    n_chunks = vox2d.shape[0] // (_NC * _NS)
    mesh = plsc.VectorSubcoreMesh(core_axis_name="c", subcore_axis_name="s")
    k = pl.kernel(
        _scatter_body,
        out_type=jax.ShapeDtypeStruct((_NC, _OUT_ROWS, f), jnp.float32),
        mesh=mesh,
        compiler_params=pltpu.CompilerParams(use_tc_tiling_on_sc=False),
        scratch_types=[
            pltpu.VMEM((n_chunks, _CHUNK), jnp.int32),
            pltpu.VMEM((_CHUNK, f), jnp.float32),
            pltpu.VMEM((2, _CHUNK, f), jnp.float32),
            pltpu.VMEM_SHARED((_ACC_ROWS, f), jnp.float32),
            pltpu.SemaphoreType.DMA,
            pltpu.SemaphoreType.DMA,
            pltpu.SemaphoreType.DMA,
        ],
    )
    return k(vox2d, feats2)


def _combine_body(p_ref, out_ref):
    v = p_ref[0] + p_ref[1]
    out_ref[...] = v[None, :, :, :_BOX, :]


def _combine(partials5):
    f = partials5.shape[4]
    return pl.pallas_call(
        _combine_body,
        out_shape=jax.ShapeDtypeStruct((1, _BOX, _BOX, _BOX, f), jnp.float32),
    )(partials5)


def kernel(coords, features):
    b, n, _ = coords.shape
    f = features.shape[2]
    p = b * n
    assert p % (_NC * _NS * _CHUNK) == 0

    coords3 = coords.reshape(p // _CHUNK, _CHUNK, 3).transpose(2, 0, 1)
    vox2d = _voxelize(coords3)
    partials = _scatter(vox2d, features, f)
    partials5 = partials.reshape(_NC, _BOX, _BOX, _ZPAD, f)
    batch0 = _combine(partials5)
    grid = jnp.zeros((b, _BOX, _BOX, _BOX, f), jnp.float32)
    return lax.dynamic_update_slice(grid, batch0, (0, 0, 0, 0, 0))
